# Initial kernel scaffold; baseline (speedup 1.0000x reference)
import jax
import jax.numpy as jnp
from jax.experimental import pallas as pl


def kernel(x, W_enc, W_dec, b_enc, b_dec):
    raise NotImplementedError("write your pallas kernel here")


# speedup vs baseline: 11.3250x; 11.3250x over previous
import jax
import jax.numpy as jnp
from jax.experimental import pallas as pl
from jax.experimental.pallas import tpu as pltpu

_D_MODEL = 2048
_D_SAE = 16384
_K = 64
_N_TOK = 8192

_BM_E = 1024
_BN_E = 1024
_BM_S = 256
_BM_D = 1024
_BK_D = 1024


def _sortable(pre):
    bits = jax.lax.bitcast_convert_type(pre, jnp.int32)
    return jnp.where(bits >= 0, bits, bits ^ jnp.int32(0x7FFFFFFF))


def _encode_body(x_ref, w_ref, benc_ref, bdec_ref, pre_ref):
    xc = x_ref[...] - bdec_ref[...]
    pre_ref[...] = (
        jnp.dot(xc, w_ref[...], preferred_element_type=jnp.float32)
        + benc_ref[...]
    )


def _select_body(pre_ref, t_ref):
    s = _sortable(pre_ref[...])
    rows = s.shape[0]
    p = jnp.full((rows, 1), jnp.iinfo(jnp.int32).min, jnp.int32)
    for b in range(31, -1, -1):
        inc = jnp.int32(-2147483648) if b == 31 else jnp.int32(1 << b)
        cand = p + inc
        cnt = jnp.sum((s >= cand).astype(jnp.float32), axis=1, keepdims=True)
        p = jnp.where(cnt >= jnp.float32(_K), cand, p)
    t_ref[...] = p


def _decode_body(pre_ref, t_ref, w_ref, bdec_ref, out_ref):
    k = pl.program_id(1)
    pre = pre_ref[...]
    s = _sortable(pre)
    acts = jnp.where(s >= t_ref[...], jnp.maximum(pre, 0.0), 0.0)
    contrib = jnp.dot(acts, w_ref[...], preferred_element_type=jnp.float32)

    @pl.when(k == 0)
    def _():
        out_ref[...] = contrib + bdec_ref[...]

    @pl.when(k != 0)
    def _():
        out_ref[...] += contrib


def kernel(x, W_enc, W_dec, b_enc, b_dec):
    n_tok, d_model = x.shape
    d_sae = W_enc.shape[1]
    benc2 = b_enc.reshape(1, d_sae)
    bdec2 = b_dec.reshape(1, d_model)

    pre = pl.pallas_call(
        _encode_body,
        grid=(n_tok // _BM_E, d_sae // _BN_E),
        in_specs=[
            pl.BlockSpec((_BM_E, d_model), lambda i, j: (i, 0)),
            pl.BlockSpec((d_model, _BN_E), lambda i, j: (0, j)),
            pl.BlockSpec((1, _BN_E), lambda i, j: (0, j)),
            pl.BlockSpec((1, d_model), lambda i, j: (0, 0)),
        ],
        out_specs=pl.BlockSpec((_BM_E, _BN_E), lambda i, j: (i, j)),
        out_shape=jax.ShapeDtypeStruct((n_tok, d_sae), jnp.float32),
        compiler_params=pltpu.CompilerParams(
            dimension_semantics=("parallel", "parallel"),
        ),
    )(x, W_enc, benc2, bdec2)

    t = pl.pallas_call(
        _select_body,
        grid=(n_tok // _BM_S,),
        in_specs=[pl.BlockSpec((_BM_S, d_sae), lambda i: (i, 0))],
        out_specs=pl.BlockSpec((_BM_S, 1), lambda i: (i, 0)),
        out_shape=jax.ShapeDtypeStruct((n_tok, 1), jnp.int32),
        compiler_params=pltpu.CompilerParams(
            dimension_semantics=("parallel",),
        ),
    )(pre)

    out = pl.pallas_call(
        _decode_body,
        grid=(n_tok // _BM_D, d_sae // _BK_D),
        in_specs=[
            pl.BlockSpec((_BM_D, _BK_D), lambda i, k: (i, k)),
            pl.BlockSpec((_BM_D, 1), lambda i, k: (i, 0)),
            pl.BlockSpec((_BK_D, d_model), lambda i, k: (k, 0)),
            pl.BlockSpec((1, d_model), lambda i, k: (0, 0)),
        ],
        out_specs=pl.BlockSpec((_BM_D, d_model), lambda i, k: (i, 0)),
        out_shape=jax.ShapeDtypeStruct((n_tok, d_model), jnp.float32),
        compiler_params=pltpu.CompilerParams(
            dimension_semantics=("parallel", "arbitrary"),
        ),
    )(pre, t, W_dec, bdec2)
    return out
